# baseline (speedup 1.0000x reference)
import jax
import jax.numpy as jnp
from jax.experimental import pallas as pl

TOP_K = 8


def _norm(a, axis=-1, eps=1e-12):
    n = jnp.linalg.norm(a, axis=axis, keepdims=True)
    return a / jnp.maximum(n, eps)


def kernel(x, memory):
    q = _norm(x, axis=-1)
    m = _norm(memory, axis=-1)
    sim = jnp.einsum('bsd,md->bsm', q, m)
    tks, tki = jax.lax.top_k(sim, TOP_K)
    tkm = jnp.take(memory, tki, axis=0)
    w = jax.nn.softmax(tks, axis=-1)[..., None]
    feat = jnp.sum(tkm * w, axis=2)
    return x + feat


# speedup vs baseline: 15.6294x; 15.6294x over previous
import functools

import jax
import jax.numpy as jnp
from jax import lax
from jax.experimental import pallas as pl
from jax.experimental.pallas import tpu as pltpu
from jax.experimental.pallas import tpu_sc as plsc

B, S, D, M, K = 4, 2048, 1024, 8192, 8
NQALL = B * S

QT = 256
MB = 1024
NQ = NQALL // QT
NM = M // MB

NEG = float("-inf")


def _norm_body(x_ref, m_ref, xo_ref, mo_ref):
    for a_ref, o_ref in ((x_ref, xo_ref), (m_ref, mo_ref)):
        a = a_ref[...]
        n = jnp.sqrt(jnp.sum(a * a, axis=1, keepdims=True))
        o_ref[...] = (a / jnp.maximum(n, 1e-12)).astype(jnp.bfloat16)


def _normalize(x2, memory):
    nblk = 8
    rows = NQALL // nblk
    return pl.pallas_call(
        _norm_body,
        grid=(nblk,),
        in_specs=[
            pl.BlockSpec((rows, D), lambda i: (i, 0)),
            pl.BlockSpec((M // nblk, D), lambda i: (i, 0)),
        ],
        out_specs=[
            pl.BlockSpec((rows, D), lambda i: (i, 0)),
            pl.BlockSpec((M // nblk, D), lambda i: (i, 0)),
        ],
        out_shape=[
            jax.ShapeDtypeStruct((NQALL, D), jnp.bfloat16),
            jax.ShapeDtypeStruct((M, D), jnp.bfloat16),
        ],
    )(x2, memory)


def _sim_topk_body(x_ref, m_ref, wts_ref, idx_ref, sims):
    mi = pl.program_id(1)
    sblk = lax.dot_general(
        x_ref[...], m_ref[...],
        (((1,), (1,)), ((), ())),
        preferred_element_type=jnp.float32,
    )
    sims[:, pl.ds(mi * MB, MB)] = sblk

    @pl.when(mi == NM - 1)
    def _():
        iota = lax.broadcasted_iota(jnp.int32, (QT, M), 1)
        cur = sims[...]
        vals = []
        idxs = []
        for _k in range(K):
            vmax = jnp.max(cur, axis=1, keepdims=True)
            im = jnp.min(jnp.where(cur == vmax, iota, M), axis=1, keepdims=True)
            vals.append(vmax)
            idxs.append(im)
            if _k < K - 1:
                cur = jnp.where(iota == im, NEG, cur)
        ii = jnp.concatenate(idxs, axis=1)
        mx = vals[0]
        for vk in vals[1:]:
            mx = jnp.maximum(mx, vk)
        es = [jnp.exp(vk - mx) for vk in vals]
        den = es[0]
        for ek in es[1:]:
            den = den + ek
        for k in range(K):
            wts_ref[:, k, :] = jnp.broadcast_to(es[k] / den, (QT, 16))
        idx_ref[...] = ii


def _sim_topk(xn, mn):
    return pl.pallas_call(
        _sim_topk_body,
        grid=(NQ, NM),
        in_specs=[
            pl.BlockSpec((QT, D), lambda qi, mi: (qi, 0)),
            pl.BlockSpec((MB, D), lambda qi, mi: (mi, 0)),
        ],
        out_specs=[
            pl.BlockSpec((QT, K, 16), lambda qi, mi: (qi, 0, 0)),
            pl.BlockSpec((QT, K), lambda qi, mi: (qi, 0)),
        ],
        out_shape=[
            jax.ShapeDtypeStruct((NQALL, K, 16), jnp.float32),
            jax.ShapeDtypeStruct((NQALL, K), jnp.int32),
        ],
        scratch_shapes=[pltpu.VMEM((QT, M), jnp.float32)],
    )(xn, mn)


NC, NS, LN = 2, 16, 16
NW = NC * NS
QPW = NQALL // NW
G = 8
NG = QPW // G
DCH = D // LN


def _sc_body(mem_hbm, idxf_hbm, wexp_hbm, x_hbm, out_hbm,
             idx_v, w_v, rows_v, x_v, o_v, sem):
    wid = lax.axis_index("s") * NC + lax.axis_index("c")
    qbase0 = wid * QPW

    def group(g, carry):
        qb = qbase0 + g * G
        eb = qb * K
        pltpu.sync_copy(idxf_hbm.at[pl.ds(eb, G * K)], idx_v)
        pltpu.sync_copy(wexp_hbm.at[pl.ds(eb, G * K)], w_v)
        cp = pltpu.async_copy(mem_hbm.at[idx_v], rows_v, sem)
        pltpu.sync_copy(x_hbm.at[pl.ds(qb, G)], x_v)
        cp.wait()
        for q in range(G):
            w16 = [w_v[q * K + k, :] for k in range(K)]

            def dchunk(d, c, _q=q, _w=w16):
                off = d * LN
                acc = x_v[_q, pl.ds(off, LN)]
                for k in range(K):
                    acc = acc + _w[k] * rows_v[_q * K + k, pl.ds(off, LN)]
                o_v[_q, pl.ds(off, LN)] = acc
                return c

            lax.fori_loop(0, DCH, dchunk, 0)
        pltpu.sync_copy(o_v, out_hbm.at[pl.ds(qb, G)])
        return carry

    lax.fori_loop(0, NG, group, 0)


def _sc_gather(memory, idxf, wexp, x2):
    mesh = plsc.VectorSubcoreMesh(core_axis_name="c", subcore_axis_name="s")
    fn = functools.partial(
        pl.kernel,
        mesh=mesh,
        out_type=jax.ShapeDtypeStruct((NQALL, D), jnp.float32),
        scratch_types=[
            pltpu.VMEM((G * K,), jnp.int32),
            pltpu.VMEM((G * K, 16), jnp.float32),
            pltpu.VMEM((G * K, D), jnp.float32),
            pltpu.VMEM((G, D), jnp.float32),
            pltpu.VMEM((G, D), jnp.float32),
            pltpu.SemaphoreType.DMA,
        ],
    )(_sc_body)
    return fn(memory, idxf, wexp, x2)


def kernel(x, memory):
    x2 = x.reshape(NQALL, D)
    xn, mn = _normalize(x2, memory)
    wexp, idx = _sim_topk(xn, mn)
    out2 = _sc_gather(memory, idx.reshape(-1), wexp.reshape(NQALL * K, 16), x2)
    return out2.reshape(B, S, D)


# speedup vs baseline: 16.6894x; 1.0678x over previous
import functools

import jax
import jax.numpy as jnp
from jax import lax
from jax.experimental import pallas as pl
from jax.experimental.pallas import tpu as pltpu
from jax.experimental.pallas import tpu_sc as plsc

B, S, D, M, K = 4, 2048, 1024, 8192, 8
NQALL = B * S

QT = 256
MB = 1024
NQ = NQALL // QT
NM = M // MB

NEG = float("-inf")


def _norm_body(x_ref, m_ref, xo_ref, mo_ref):
    for a_ref, o_ref in ((x_ref, xo_ref), (m_ref, mo_ref)):
        a = a_ref[...]
        n = jnp.sqrt(jnp.sum(a * a, axis=1, keepdims=True))
        o_ref[...] = (a / jnp.maximum(n, 1e-12)).astype(jnp.bfloat16)


def _normalize(x2, memory):
    nblk = 8
    rows = NQALL // nblk
    return pl.pallas_call(
        _norm_body,
        grid=(nblk,),
        in_specs=[
            pl.BlockSpec((rows, D), lambda i: (i, 0)),
            pl.BlockSpec((M // nblk, D), lambda i: (i, 0)),
        ],
        out_specs=[
            pl.BlockSpec((rows, D), lambda i: (i, 0)),
            pl.BlockSpec((M // nblk, D), lambda i: (i, 0)),
        ],
        out_shape=[
            jax.ShapeDtypeStruct((NQALL, D), jnp.bfloat16),
            jax.ShapeDtypeStruct((M, D), jnp.bfloat16),
        ],
    )(x2, memory)


def _sim_topk_body(x_ref, m_ref, wts_ref, idx_ref, sims):
    mi = pl.program_id(1)
    sblk = lax.dot_general(
        x_ref[...], m_ref[...],
        (((1,), (1,)), ((), ())),
        preferred_element_type=jnp.float32,
    )
    sims[:, pl.ds(mi * MB, MB)] = sblk

    @pl.when(mi == NM - 1)
    def _():
        NB = M // 128
        bm = [
            jnp.max(sims[:, b * 128:(b + 1) * 128], axis=1, keepdims=True)
            for b in range(NB)
        ]
        BM = jnp.concatenate(bm, axis=1)
        iob = lax.broadcasted_iota(jnp.int32, (QT, NB), 1)
        curb = BM
        bsel = []
        for j in range(K):
            m = jnp.max(curb, axis=1, keepdims=True)
            bj = jnp.min(jnp.where(curb == m, iob, NB), axis=1, keepdims=True)
            bsel.append(bj)
            if j < K - 1:
                curb = jnp.where(iob == bj, NEG, curb)
        zero = jnp.zeros((QT, 128), jnp.float32)
        cands = []
        for j in range(K):
            acc = zero
            for b in range(NB):
                acc = acc + jnp.where(
                    bsel[j] == b, sims[:, b * 128:(b + 1) * 128], 0.0)
            cands.append(acc)
        C = jnp.concatenate(cands, axis=1)
        io128 = lax.broadcasted_iota(jnp.int32, (QT, 128), 1)
        GI = jnp.concatenate(
            [bsel[j] * 128 + io128 for j in range(K)], axis=1)
        cur = C
        vals = []
        idxs = []
        for _k in range(K):
            vmax = jnp.max(cur, axis=1, keepdims=True)
            im = jnp.min(jnp.where(cur == vmax, GI, M), axis=1, keepdims=True)
            vals.append(vmax)
            idxs.append(im)
            if _k < K - 1:
                cur = jnp.where((cur == vmax) & (GI == im), NEG, cur)
        ii = jnp.concatenate(idxs, axis=1)
        mx = vals[0]
        for vk in vals[1:]:
            mx = jnp.maximum(mx, vk)
        es = [jnp.exp(vk - mx) for vk in vals]
        den = es[0]
        for ek in es[1:]:
            den = den + ek
        for k in range(K):
            wts_ref[:, k, :] = jnp.broadcast_to(es[k] / den, (QT, 16))
        idx_ref[...] = ii


def _sim_topk(xn, mn):
    return pl.pallas_call(
        _sim_topk_body,
        grid=(NQ, NM),
        in_specs=[
            pl.BlockSpec((QT, D), lambda qi, mi: (qi, 0)),
            pl.BlockSpec((MB, D), lambda qi, mi: (mi, 0)),
        ],
        out_specs=[
            pl.BlockSpec((QT, K, 16), lambda qi, mi: (qi, 0, 0)),
            pl.BlockSpec((QT, K), lambda qi, mi: (qi, 0)),
        ],
        out_shape=[
            jax.ShapeDtypeStruct((NQALL, K, 16), jnp.float32),
            jax.ShapeDtypeStruct((NQALL, K), jnp.int32),
        ],
        scratch_shapes=[pltpu.VMEM((QT, M), jnp.float32)],
    )(xn, mn)


NC, NS, LN = 2, 16, 16
NW = NC * NS
QPW = NQALL // NW
G = 8
NG = QPW // G
DCH = D // LN


def _sc_body(mem_hbm, idxf_hbm, wexp_hbm, x_hbm, out_hbm,
             idx_v, w_v, rows_v, x_v, o_v, sem):
    wid = lax.axis_index("s") * NC + lax.axis_index("c")
    qbase0 = wid * QPW

    def group(g, carry):
        qb = qbase0 + g * G
        eb = qb * K
        pltpu.sync_copy(idxf_hbm.at[pl.ds(eb, G * K)], idx_v)
        pltpu.sync_copy(wexp_hbm.at[pl.ds(eb, G * K)], w_v)
        cp = pltpu.async_copy(mem_hbm.at[idx_v], rows_v, sem)
        pltpu.sync_copy(x_hbm.at[pl.ds(qb, G)], x_v)
        cp.wait()
        for q in range(G):
            w16 = [w_v[q * K + k, :] for k in range(K)]

            def dchunk(d, c, _q=q, _w=w16):
                off = d * LN
                acc = x_v[_q, pl.ds(off, LN)]
                for k in range(K):
                    acc = acc + _w[k] * rows_v[_q * K + k, pl.ds(off, LN)]
                o_v[_q, pl.ds(off, LN)] = acc
                return c

            lax.fori_loop(0, DCH, dchunk, 0)
        pltpu.sync_copy(o_v, out_hbm.at[pl.ds(qb, G)])
        return carry

    lax.fori_loop(0, NG, group, 0)


def _sc_gather(memory, idxf, wexp, x2):
    mesh = plsc.VectorSubcoreMesh(core_axis_name="c", subcore_axis_name="s")
    fn = functools.partial(
        pl.kernel,
        mesh=mesh,
        out_type=jax.ShapeDtypeStruct((NQALL, D), jnp.float32),
        scratch_types=[
            pltpu.VMEM((G * K,), jnp.int32),
            pltpu.VMEM((G * K, 16), jnp.float32),
            pltpu.VMEM((G * K, D), jnp.float32),
            pltpu.VMEM((G, D), jnp.float32),
            pltpu.VMEM((G, D), jnp.float32),
            pltpu.SemaphoreType.DMA,
        ],
    )(_sc_body)
    return fn(memory, idxf, wexp, x2)


def kernel(x, memory):
    x2 = x.reshape(NQALL, D)
    xn, mn = _normalize(x2, memory)
    wexp, idx = _sim_topk(xn, mn)
    out2 = _sc_gather(memory, idx.reshape(-1), wexp.reshape(NQALL * K, 16), x2)
    return out2.reshape(B, S, D)


# speedup vs baseline: 19.2360x; 1.1526x over previous
import functools

import jax
import jax.numpy as jnp
from jax import lax
from jax.experimental import pallas as pl
from jax.experimental.pallas import tpu as pltpu
from jax.experimental.pallas import tpu_sc as plsc

B, S, D, M, K = 4, 2048, 1024, 8192, 8
NQALL = B * S

QT = 512
MB = 1024
NQ = NQALL // QT
NM = M // MB

NEG = float("-inf")


def _norm_body(x_ref, m_ref, xo_ref, mo_ref):
    for a_ref, o_ref in ((x_ref, xo_ref), (m_ref, mo_ref)):
        a = a_ref[...]
        n = jnp.sqrt(jnp.sum(a * a, axis=1, keepdims=True))
        o_ref[...] = (a / jnp.maximum(n, 1e-12)).astype(jnp.bfloat16)


def _normalize(x2, memory):
    nblk = 8
    rows = NQALL // nblk
    return pl.pallas_call(
        _norm_body,
        grid=(nblk,),
        in_specs=[
            pl.BlockSpec((rows, D), lambda i: (i, 0)),
            pl.BlockSpec((M // nblk, D), lambda i: (i, 0)),
        ],
        out_specs=[
            pl.BlockSpec((rows, D), lambda i: (i, 0)),
            pl.BlockSpec((M // nblk, D), lambda i: (i, 0)),
        ],
        out_shape=[
            jax.ShapeDtypeStruct((NQALL, D), jnp.bfloat16),
            jax.ShapeDtypeStruct((M, D), jnp.bfloat16),
        ],
    )(x2, memory)


def _sim_topk_body(x_ref, m_ref, wts_ref, idx_ref, sims):
    mi = pl.program_id(1)
    sblk = lax.dot_general(
        x_ref[...], m_ref[...],
        (((1,), (1,)), ((), ())),
        preferred_element_type=jnp.float32,
    )
    sims[:, pl.ds(mi * MB, MB)] = sblk

    @pl.when(mi == NM - 1)
    def _():
        NB = M // 128
        bm = [
            jnp.max(sims[:, b * 128:(b + 1) * 128], axis=1, keepdims=True)
            for b in range(NB)
        ]
        BM = jnp.concatenate(bm, axis=1)
        iob = lax.broadcasted_iota(jnp.int32, (QT, NB), 1)
        curb = BM
        bsel = []
        for j in range(K):
            m = jnp.max(curb, axis=1, keepdims=True)
            bj = jnp.min(jnp.where(curb == m, iob, NB), axis=1, keepdims=True)
            bsel.append(bj)
            if j < K - 1:
                curb = jnp.where(iob == bj, NEG, curb)
        zero = jnp.zeros((QT, 128), jnp.float32)
        cands = []
        for j in range(K):
            acc = zero
            for b in range(NB):
                acc = acc + jnp.where(
                    bsel[j] == b, sims[:, b * 128:(b + 1) * 128], 0.0)
            cands.append(acc)
        C = jnp.concatenate(cands, axis=1)
        io128 = lax.broadcasted_iota(jnp.int32, (QT, 128), 1)
        GI = jnp.concatenate(
            [bsel[j] * 128 + io128 for j in range(K)], axis=1)
        cur = C
        vals = []
        idxs = []
        for _k in range(K):
            vmax = jnp.max(cur, axis=1, keepdims=True)
            im = jnp.min(jnp.where(cur == vmax, GI, M), axis=1, keepdims=True)
            vals.append(vmax)
            idxs.append(im)
            if _k < K - 1:
                cur = jnp.where((cur == vmax) & (GI == im), NEG, cur)
        ii = jnp.concatenate(idxs, axis=1)
        mx = vals[0]
        for vk in vals[1:]:
            mx = jnp.maximum(mx, vk)
        es = [jnp.exp(vk - mx) for vk in vals]
        den = es[0]
        for ek in es[1:]:
            den = den + ek
        for k in range(K):
            wts_ref[:, k, :] = jnp.broadcast_to(es[k] / den, (QT, 16))
        idx_ref[...] = ii


def _sim_topk(xn, mn):
    return pl.pallas_call(
        _sim_topk_body,
        grid=(NQ, NM),
        in_specs=[
            pl.BlockSpec((QT, D), lambda qi, mi: (qi, 0)),
            pl.BlockSpec((MB, D), lambda qi, mi: (mi, 0)),
        ],
        out_specs=[
            pl.BlockSpec((QT, K, 16), lambda qi, mi: (qi, 0, 0)),
            pl.BlockSpec((QT, K), lambda qi, mi: (qi, 0)),
        ],
        out_shape=[
            jax.ShapeDtypeStruct((NQALL, K, 16), jnp.float32),
            jax.ShapeDtypeStruct((NQALL, K), jnp.int32),
        ],
        scratch_shapes=[pltpu.VMEM((QT, M), jnp.float32)],
    )(xn, mn)


NC, NS, LN = 2, 16, 16
NW = NC * NS
QPW = NQALL // NW
G = 8
NG = QPW // G
DCH = D // LN


def _sc_body(mem_hbm, idxf_hbm, wexp_hbm, x_hbm, out_hbm,
             idx_v, w_v, rows_v, x_v, o_v, sem):
    wid = lax.axis_index("s") * NC + lax.axis_index("c")
    qbase0 = wid * QPW

    def group(g, carry):
        qb = qbase0 + g * G
        eb = qb * K
        pltpu.sync_copy(idxf_hbm.at[pl.ds(eb, G * K)], idx_v)
        pltpu.sync_copy(wexp_hbm.at[pl.ds(eb, G * K)], w_v)
        cp = pltpu.async_copy(mem_hbm.at[idx_v], rows_v, sem)
        pltpu.sync_copy(x_hbm.at[pl.ds(qb, G)], x_v)
        cp.wait()
        for q in range(G):
            w16 = [w_v[q * K + k, :] for k in range(K)]

            def dchunk(d, c, _q=q, _w=w16):
                off = d * LN
                acc = x_v[_q, pl.ds(off, LN)]
                for k in range(K):
                    acc = acc + _w[k] * rows_v[_q * K + k, pl.ds(off, LN)]
                o_v[_q, pl.ds(off, LN)] = acc
                return c

            lax.fori_loop(0, DCH, dchunk, 0)
        pltpu.sync_copy(o_v, out_hbm.at[pl.ds(qb, G)])
        return carry

    lax.fori_loop(0, NG, group, 0)


def _sc_gather(memory, idxf, wexp, x2):
    mesh = plsc.VectorSubcoreMesh(core_axis_name="c", subcore_axis_name="s")
    fn = functools.partial(
        pl.kernel,
        mesh=mesh,
        out_type=jax.ShapeDtypeStruct((NQALL, D), jnp.float32),
        scratch_types=[
            pltpu.VMEM((G * K,), jnp.int32),
            pltpu.VMEM((G * K, 16), jnp.float32),
            pltpu.VMEM((G * K, D), jnp.float32),
            pltpu.VMEM((G, D), jnp.float32),
            pltpu.VMEM((G, D), jnp.float32),
            pltpu.SemaphoreType.DMA,
        ],
    )(_sc_body)
    return fn(memory, idxf, wexp, x2)


def kernel(x, memory):
    x2 = x.reshape(NQALL, D)
    xn, mn = _normalize(x2, memory)
    wexp, idx = _sim_topk(xn, mn)
    out2 = _sc_gather(memory, idx.reshape(-1), wexp.reshape(NQALL * K, 16), x2)
    return out2.reshape(B, S, D)


# speedup vs baseline: 19.7598x; 1.0272x over previous
import functools

import jax
import jax.numpy as jnp
from jax import lax
from jax.experimental import pallas as pl
from jax.experimental.pallas import tpu as pltpu
from jax.experimental.pallas import tpu_sc as plsc

B, S, D, M, K = 4, 2048, 1024, 8192, 8
NQALL = B * S

QT = 512
MB = 1024
NQ = NQALL // QT
NM = M // MB

NEG = float("-inf")


def _norm_body(x_ref, m_ref, xo_ref, mo_ref):
    for a_ref, o_ref in ((x_ref, xo_ref), (m_ref, mo_ref)):
        a = a_ref[...]
        n = jnp.sqrt(jnp.sum(a * a, axis=1, keepdims=True))
        o_ref[...] = (a / jnp.maximum(n, 1e-12)).astype(jnp.bfloat16)


def _normalize(x2, memory):
    nblk = 8
    rows = NQALL // nblk
    return pl.pallas_call(
        _norm_body,
        grid=(nblk,),
        in_specs=[
            pl.BlockSpec((rows, D), lambda i: (i, 0)),
            pl.BlockSpec((M // nblk, D), lambda i: (i, 0)),
        ],
        out_specs=[
            pl.BlockSpec((rows, D), lambda i: (i, 0)),
            pl.BlockSpec((M // nblk, D), lambda i: (i, 0)),
        ],
        out_shape=[
            jax.ShapeDtypeStruct((NQALL, D), jnp.bfloat16),
            jax.ShapeDtypeStruct((M, D), jnp.bfloat16),
        ],
    )(x2, memory)


def _sim_topk_body(x_ref, m_ref, wts_ref, idx_ref, sims):
    mi = pl.program_id(1)
    sblk = lax.dot_general(
        x_ref[...], m_ref[...],
        (((1,), (1,)), ((), ())),
        preferred_element_type=jnp.float32,
    )
    sims[:, pl.ds(mi * MB, MB)] = sblk

    @pl.when(mi == NM - 1)
    def _():
        NB = M // 128
        bm = [
            jnp.max(sims[:, b * 128:(b + 1) * 128], axis=1, keepdims=True)
            for b in range(NB)
        ]
        BM = jnp.concatenate(bm, axis=1)
        iob = lax.broadcasted_iota(jnp.int32, (QT, NB), 1)
        curb = BM
        bsel = []
        for j in range(K):
            m = jnp.max(curb, axis=1, keepdims=True)
            bj = jnp.min(jnp.where(curb == m, iob, NB), axis=1, keepdims=True)
            bsel.append(bj)
            if j < K - 1:
                curb = jnp.where(iob == bj, NEG, curb)
        zero = jnp.zeros((QT, 128), jnp.float32)
        cands = []
        for j in range(K):
            acc = zero
            for b in range(NB):
                acc = acc + jnp.where(
                    bsel[j] == b, sims[:, b * 128:(b + 1) * 128], 0.0)
            cands.append(acc)
        C = jnp.concatenate(cands, axis=1)
        io128 = lax.broadcasted_iota(jnp.int32, (QT, 128), 1)
        GI = jnp.concatenate(
            [bsel[j] * 128 + io128 for j in range(K)], axis=1)
        cur = C
        vals = []
        idxs = []
        for _k in range(K):
            vmax = jnp.max(cur, axis=1, keepdims=True)
            im = jnp.min(jnp.where(cur == vmax, GI, M), axis=1, keepdims=True)
            vals.append(vmax)
            idxs.append(im)
            if _k < K - 1:
                cur = jnp.where((cur == vmax) & (GI == im), NEG, cur)
        ii = jnp.concatenate(idxs, axis=1)
        mx = vals[0]
        for vk in vals[1:]:
            mx = jnp.maximum(mx, vk)
        es = [jnp.exp(vk - mx) for vk in vals]
        den = es[0]
        for ek in es[1:]:
            den = den + ek
        for k in range(K):
            wts_ref[:, k, :] = jnp.broadcast_to(es[k] / den, (QT, 16))
        idx_ref[...] = ii


def _sim_topk(xn, mn):
    return pl.pallas_call(
        _sim_topk_body,
        grid=(NQ, NM),
        in_specs=[
            pl.BlockSpec((QT, D), lambda qi, mi: (qi, 0)),
            pl.BlockSpec((MB, D), lambda qi, mi: (mi, 0)),
        ],
        out_specs=[
            pl.BlockSpec((QT, K, 16), lambda qi, mi: (qi, 0, 0)),
            pl.BlockSpec((QT, K), lambda qi, mi: (qi, 0)),
        ],
        out_shape=[
            jax.ShapeDtypeStruct((NQALL, K, 16), jnp.float32),
            jax.ShapeDtypeStruct((NQALL, K), jnp.int32),
        ],
        scratch_shapes=[pltpu.VMEM((QT, M), jnp.float32)],
    )(xn, mn)


NC, NS, LN = 2, 16, 16
NW = NC * NS
QPW = NQALL // NW
G = 4
NG = QPW // G
DCH = D // LN


def _sc_body(mem_hbm, idxf_hbm, wexp_hbm, x_hbm, out_hbm,
             idx_v, w_v, rows_v, x_v, o_v,
             sem_i0, sem_i1, sem_w0, sem_w1, sem_x0, sem_x1,
             sem_g0, sem_g1, sem_o0, sem_o1):
    wid = lax.axis_index("s") * NC + lax.axis_index("c")
    qbase0 = wid * QPW
    sem_i = (sem_i0, sem_i1)
    sem_w = (sem_w0, sem_w1)
    sem_x = (sem_x0, sem_x1)
    sem_g = (sem_g0, sem_g1)
    sem_o = (sem_o0, sem_o1)

    def _loads(g, buf):
        qb = qbase0 + g * G
        eb = qb * K
        pltpu.async_copy(idxf_hbm.at[pl.ds(eb, G * K)], idx_v.at[buf], sem_i[buf])
        pltpu.async_copy(wexp_hbm.at[pl.ds(eb, G * K)], w_v.at[buf], sem_w[buf])
        pltpu.async_copy(x_hbm.at[pl.ds(qb, G)], x_v.at[buf], sem_x[buf])

    def _wait_loads(g, buf):
        qb = qbase0 + g * G
        eb = qb * K
        pltpu.make_async_copy(wexp_hbm.at[pl.ds(eb, G * K)], w_v.at[buf], sem_w[buf]).wait()
        pltpu.make_async_copy(x_hbm.at[pl.ds(qb, G)], x_v.at[buf], sem_x[buf]).wait()

    def _gather(g, buf):
        eb = (qbase0 + g * G) * K
        pltpu.make_async_copy(idxf_hbm.at[pl.ds(eb, G * K)], idx_v.at[buf], sem_i[buf]).wait()
        pltpu.async_copy(mem_hbm.at[idx_v.at[buf]], rows_v.at[buf], sem_g[buf])

    def _wait_gather(g, buf):
        pltpu.make_async_copy(mem_hbm.at[idx_v.at[buf]], rows_v.at[buf], sem_g[buf]).wait()

    def _out_start(g, buf):
        qb = qbase0 + g * G
        pltpu.async_copy(o_v.at[buf], out_hbm.at[pl.ds(qb, G)], sem_o[buf])

    def _out_wait(g, buf):
        qb = qbase0 + g * G
        pltpu.make_async_copy(o_v.at[buf], out_hbm.at[pl.ds(qb, G)], sem_o[buf]).wait()

    def _compute(buf):
        for q in range(G):
            w16 = [w_v[buf, q * K + k, :] for k in range(K)]

            def dchunk(d, c, _q=q, _w=w16):
                off = d * LN
                acc = x_v[buf, _q, pl.ds(off, LN)]
                for k in range(K):
                    acc = acc + _w[k] * rows_v[buf, _q * K + k, pl.ds(off, LN)]
                o_v[buf, _q, pl.ds(off, LN)] = acc
                return c

            lax.fori_loop(0, DCH, dchunk, 0)

    _loads(0, 0)
    _gather(0, 0)
    _loads(1, 1)

    def pair(p, carry):
        for sub in range(2):
            g = 2 * p + sub
            buf = sub
            nbuf = 1 - sub

            @pl.when(g + 1 < NG)
            def _():
                _gather(g + 1, nbuf)

            _wait_loads(g, buf)
            _wait_gather(g, buf)

            @pl.when(g >= 2)
            def _():
                _out_wait(g - 2, buf)

            _compute(buf)
            _out_start(g, buf)

            @pl.when(g + 2 < NG)
            def _():
                _loads(g + 2, buf)
        return carry

    lax.fori_loop(0, NG // 2, pair, 0)
    _out_wait(NG - 2, 0)
    _out_wait(NG - 1, 1)


def _sc_gather(memory, idxf, wexp, x2):
    mesh = plsc.VectorSubcoreMesh(core_axis_name="c", subcore_axis_name="s")
    fn = functools.partial(
        pl.kernel,
        mesh=mesh,
        out_type=jax.ShapeDtypeStruct((NQALL, D), jnp.float32),
        scratch_types=[
            pltpu.VMEM((2, G * K), jnp.int32),
            pltpu.VMEM((2, G * K, 16), jnp.float32),
            pltpu.VMEM((2, G * K, D), jnp.float32),
            pltpu.VMEM((2, G, D), jnp.float32),
            pltpu.VMEM((2, G, D), jnp.float32),
        ] + [pltpu.SemaphoreType.DMA] * 10,
    )(_sc_body)
    return fn(memory, idxf, wexp, x2)


def kernel(x, memory):
    x2 = x.reshape(NQALL, D)
    xn, mn = _normalize(x2, memory)
    wexp, idx = _sim_topk(xn, mn)
    out2 = _sc_gather(memory, idx.reshape(-1), wexp.reshape(NQALL * K, 16), x2)
    return out2.reshape(B, S, D)
